# Initial kernel scaffold; baseline (speedup 1.0000x reference)
import jax
import jax.numpy as jnp
from jax.experimental import pallas as pl


def kernel(transmission, susceptibility, is_infected, infection_time, uniform_noise, log_beta, group_ids):
    raise NotImplementedError("write your pallas kernel here")


# speedup vs baseline: 26.7865x; 26.7865x over previous
import functools

import jax
import jax.numpy as jnp
from jax import lax
from jax.experimental import pallas as pl
from jax.experimental.pallas import tpu as pltpu
from jax.experimental.pallas import tpu_sc as plsc

N_AGENTS = 1000000
N_GROUPS = 100000
TAU = 0.1
DELTA_T = 1.0
NOW = 0.5
EPS = 1e-9

LANES = 128
ROWS = 7936
NP = ROWS * LANES
NSC = 2
NTILE = 16
NW = NSC * NTILE
SC_ROWS = ROWS // NTILE
W_ROWS = ROWS // NW
SCAT_CH = 16
SCAT_NCH = SC_ROWS // SCAT_CH
GATH_CH = 8
GATH_NCH = W_ROWS // GATH_CH
NGP = 100352
GSLICE = NGP // NTILE

BLK_R = 32
TC_GRID = ROWS // BLK_R


def _sc_segment_gather(trans2d, gid2d, zeros_g):
    mesh = plsc.VectorSubcoreMesh(core_axis_name="c", subcore_axis_name="s")

    @functools.partial(
        pl.kernel,
        out_type=jax.ShapeDtypeStruct((ROWS, LANES), jnp.float32),
        mesh=mesh,
        scratch_types=[
            pltpu.VMEM_SHARED((NGP,), jnp.float32),
            pltpu.VMEM((SCAT_CH, LANES), jnp.int32),
            pltpu.VMEM((SCAT_CH, LANES), jnp.float32),
            pltpu.VMEM((GATH_CH, LANES), jnp.float32),
            pltpu.SemaphoreType.DMA,
        ],
    )
    def k(trans_hbm, gid_hbm, zeros_hbm, gt_hbm, acc, idx_v, val_v, out_v, sem):
        c = lax.axis_index("c")
        s = lax.axis_index("s")

        pltpu.sync_copy(zeros_hbm.at[pl.ds(s * GSLICE, GSLICE)],
                        acc.at[pl.ds(s * GSLICE, GSLICE)])
        plsc.subcore_barrier()

        row0 = s * SC_ROWS

        @pl.loop(0, SCAT_NCH)
        def _(ch):
            r = row0 + ch * SCAT_CH
            pltpu.sync_copy(gid_hbm.at[pl.ds(r, SCAT_CH)], idx_v)
            pltpu.sync_copy(trans_hbm.at[pl.ds(r, SCAT_CH)], val_v)
            handles = [
                pltpu.async_copy(val_v.at[j], acc.at[idx_v.at[j]], sem,
                                 add=True)
                for j in range(SCAT_CH)
            ]
            for h in handles:
                h.wait()

        plsc.subcore_barrier()

        w = c * NTILE + s
        grow0 = w * W_ROWS

        @pl.loop(0, GATH_NCH)
        def _(ch):
            r = grow0 + ch * GATH_CH
            pltpu.sync_copy(gid_hbm.at[pl.ds(r, GATH_CH)],
                            idx_v.at[pl.ds(0, GATH_CH)])
            handles = [
                pltpu.async_copy(acc.at[idx_v.at[j]], out_v.at[j], sem)
                for j in range(GATH_CH)
            ]
            for h in handles:
                h.wait()
            pltpu.sync_copy(out_v, gt_hbm.at[pl.ds(r, GATH_CH)])

    return k(trans2d, gid2d, zeros_g)


def _tc_body(lb_ref, gt_ref, su_ref, u0_ref, u1_ref, ii_ref, it_ref, o_ref):
    beta = jnp.exp(jnp.full((BLK_R, LANES), lb_ref[0], jnp.float32))
    gt = gt_ref[...]
    su = su_ref[...]
    lam = beta * gt * su * DELTA_T
    pn = jnp.exp(-lam)
    l0 = jnp.log(pn + EPS)
    l1 = jnp.log(1.0 - pn + EPS)
    g0 = -jnp.log(-jnp.log(u0_ref[...] + EPS) + EPS)
    g1 = -jnp.log(-jnp.log(u1_ref[...] + EPS) + EPS)
    inf = jnp.where(l1 + g1 > l0 + g0, 1.0, 0.0).astype(jnp.float32)
    o_ref[0] = inf
    o_ref[1] = jnp.maximum(0.0, su - inf)
    o_ref[2] = ii_ref[...] + inf
    o_ref[3] = jnp.where(inf > 0.5, NOW, it_ref[...])


def _tc_elementwise(log_beta, gt2d, su2d, u02d, u12d, ii2d, it2d):
    blk = lambda: pl.BlockSpec((BLK_R, LANES), lambda i: (i, 0))
    return pl.pallas_call(
        _tc_body,
        grid=(TC_GRID,),
        in_specs=[
            pl.BlockSpec(memory_space=pltpu.SMEM),
            blk(), blk(), blk(), blk(), blk(), blk(),
        ],
        out_specs=pl.BlockSpec((4, BLK_R, LANES), lambda i: (0, i, 0)),
        out_shape=jax.ShapeDtypeStruct((4, ROWS, LANES), jnp.float32),
    )(log_beta, gt2d, su2d, u02d, u12d, ii2d, it2d)


def _pad2d(x):
    return jnp.pad(x, (0, NP - N_AGENTS)).reshape(ROWS, LANES)


def kernel(transmission, susceptibility, is_infected, infection_time,
           uniform_noise, log_beta, group_ids):
    trans2d = _pad2d(transmission)
    gid2d = _pad2d(group_ids)
    zeros_g = jnp.zeros((NGP,), jnp.float32)
    gt2d = _sc_segment_gather(trans2d, gid2d, zeros_g)

    su2d = _pad2d(susceptibility)
    u02d = _pad2d(uniform_noise[0])
    u12d = _pad2d(uniform_noise[1])
    ii2d = _pad2d(is_infected)
    it2d = _pad2d(infection_time)
    out3 = _tc_elementwise(log_beta, gt2d, su2d, u02d, u12d, ii2d, it2d)
    return out3.reshape(4, NP)[:, :N_AGENTS]


# speedup vs baseline: 67.6018x; 2.5237x over previous
import dataclasses
import functools

import jax
import jax.numpy as jnp
from jax import lax
from jax.experimental import pallas as pl
from jax.experimental.pallas import tpu as pltpu
from jax.experimental.pallas import tpu_sc as plsc

N_AGENTS = 1000000
N_GROUPS = 100000
TAU = 0.1
DELTA_T = 1.0
NOW = 0.5
EPS = 1e-9

LANES = 128
ROWS = 7936
NP = ROWS * LANES
NSC = 2
NTILE = 16
NW = NSC * NTILE
SC_ROWS = ROWS // NTILE
W_ROWS = ROWS // NW
SCAT_CH = 16
SCAT_NCH = SC_ROWS // SCAT_CH
GATH_CH = 8
GATH_NCH = W_ROWS // GATH_CH
NGP = 100352
GSLICE = NGP // NTILE

BLK = 32768
TC_GRID = NP // BLK


def _sc_segment_gather(trans2d, gid2d, zeros_g):
    mesh = plsc.VectorSubcoreMesh(core_axis_name="c", subcore_axis_name="s")
    cp = pltpu.CompilerParams()
    if "needs_layout_passes" in pltpu.CompilerParams.__dataclass_fields__:
        cp = dataclasses.replace(cp, needs_layout_passes=False)

    @functools.partial(
        pl.kernel,
        out_type=jax.ShapeDtypeStruct((NP,), jnp.float32),
        mesh=mesh,
        compiler_params=cp,
        scratch_types=[
            pltpu.VMEM_SHARED((NGP,), jnp.float32),
            pltpu.VMEM((NGP,), jnp.float32),
            pltpu.VMEM((SCAT_CH, LANES), jnp.int32),
            pltpu.VMEM((SCAT_CH, LANES), jnp.int32),
            pltpu.VMEM((SCAT_CH, LANES), jnp.float32),
            pltpu.VMEM((SCAT_CH, LANES), jnp.float32),
            pltpu.VMEM((GATH_CH, LANES), jnp.int32),
            pltpu.VMEM((GATH_CH * LANES,), jnp.float32),
            pltpu.SemaphoreType.DMA,
            pltpu.SemaphoreType.DMA,
            pltpu.SemaphoreType.DMA,
            pltpu.SemaphoreType.DMA,
            pltpu.SemaphoreType.DMA,
        ],
    )
    def k(trans_hbm, gid_hbm, zeros_hbm, gt_hbm, acc, tot_v,
          idx0, idx1, val0, val1, gidx, gout,
          s_st0, s_st1, s_sc0, s_sc1, s_g):
        c = lax.axis_index("c")
        s = lax.axis_index("s")
        row0 = s * SC_ROWS

        def stage_start(ch, idxb, valb, semb):
            r = row0 + ch * SCAT_CH
            pltpu.async_copy(gid_hbm.at[pl.ds(r, SCAT_CH)], idxb, semb)
            pltpu.async_copy(trans_hbm.at[pl.ds(r, SCAT_CH)], valb, semb)

        def stage_wait(idxb, valb, semb):
            pltpu.make_async_copy(gid_hbm.at[pl.ds(row0, SCAT_CH)], idxb,
                                  semb).wait()
            pltpu.make_async_copy(trans_hbm.at[pl.ds(row0, SCAT_CH)], valb,
                                  semb).wait()

        def fire(idxb, valb, semc):
            for j in range(SCAT_CH):
                pltpu.async_copy(valb.at[j], acc.at[idxb.at[j]], semc,
                                 add=True)

        def drain(idxb, valb, semc):
            for j in range(SCAT_CH):
                pltpu.make_async_copy(valb.at[j], acc.at[idxb.at[j]],
                                      semc).wait()

        stage_start(0, idx0, val0, s_st0)
        stage_start(1, idx1, val1, s_st1)
        pltpu.sync_copy(zeros_hbm.at[pl.ds(s * GSLICE, GSLICE)],
                        acc.at[pl.ds(s * GSLICE, GSLICE)])
        plsc.subcore_barrier()

        @pl.loop(0, (SCAT_NCH - 1) // 2)
        def _(i):
            ca = 2 * i
            cb = 2 * i + 1
            stage_wait(idx0, val0, s_st0)
            fire(idx0, val0, s_sc0)
            stage_wait(idx1, val1, s_st1)
            fire(idx1, val1, s_sc1)
            drain(idx0, val0, s_sc0)
            stage_start(ca + 2, idx0, val0, s_st0)
            drain(idx1, val1, s_sc1)

            @pl.when(cb + 2 < SCAT_NCH)
            def _():
                stage_start(cb + 2, idx1, val1, s_st1)

        stage_wait(idx0, val0, s_st0)
        fire(idx0, val0, s_sc0)
        drain(idx0, val0, s_sc0)
        plsc.subcore_barrier()

        pltpu.sync_copy(acc, tot_v)
        w = c * NTILE + s
        grow0 = w * W_ROWS

        @pl.loop(0, GATH_NCH)
        def _(ch):
            r = grow0 + ch * GATH_CH
            pltpu.sync_copy(gid_hbm.at[pl.ds(r, GATH_CH)], gidx)
            for j in range(GATH_CH):
                for g in range(LANES // 16):
                    idx = gidx[j, pl.ds(g * 16, 16)]
                    vals = plsc.load_gather(tot_v, [idx])
                    gout[pl.ds(j * LANES + g * 16, 16)] = vals
            pltpu.sync_copy(gout, gt_hbm.at[pl.ds(r * LANES, GATH_CH * LANES)])

    return k(trans2d, gid2d, zeros_g)


def _tc_body(lb_ref, gt_ref, su_ref, un_ref, ii_ref, it_ref, o_ref):
    beta = jnp.exp(jnp.full((BLK,), lb_ref[0], jnp.float32))
    gt = gt_ref[...]
    su = su_ref[...]
    lam = beta * gt * su * DELTA_T
    pn = jnp.exp(-lam)
    l0 = jnp.log(pn + EPS)
    l1 = jnp.log(1.0 - pn + EPS)
    g0 = -jnp.log(-jnp.log(un_ref[0] + EPS) + EPS)
    g1 = -jnp.log(-jnp.log(un_ref[1] + EPS) + EPS)
    inf = jnp.where(l1 + g1 > l0 + g0, 1.0, 0.0).astype(jnp.float32)
    o_ref[0] = inf
    o_ref[1] = jnp.maximum(0.0, su - inf)
    o_ref[2] = ii_ref[...] + inf
    o_ref[3] = jnp.where(inf > 0.5, NOW, it_ref[...])


def _tc_elementwise(log_beta, gt, su, un, ii, it):
    blk1 = lambda: pl.BlockSpec((BLK,), lambda i: (i,))
    return pl.pallas_call(
        _tc_body,
        grid=(TC_GRID,),
        in_specs=[
            pl.BlockSpec(memory_space=pltpu.SMEM),
            blk1(),
            blk1(),
            pl.BlockSpec((2, BLK), lambda i: (0, i)),
            blk1(),
            blk1(),
        ],
        out_specs=pl.BlockSpec((4, BLK), lambda i: (0, i)),
        out_shape=jax.ShapeDtypeStruct((4, N_AGENTS), jnp.float32),
    )(log_beta, gt, su, un, ii, it)


def kernel(transmission, susceptibility, is_infected, infection_time,
           uniform_noise, log_beta, group_ids):
    npad = NP - N_AGENTS
    trans2d = jnp.pad(transmission, (0, npad)).reshape(ROWS, LANES)
    tail = (N_GROUPS
            + jnp.mod(jnp.arange(npad, dtype=jnp.int32), NGP - N_GROUPS))
    gid2d = jnp.concatenate([group_ids.astype(jnp.int32), tail])
    gid2d = gid2d.reshape(ROWS, LANES)
    zeros_g = jnp.zeros((NGP,), jnp.float32)
    gt = _sc_segment_gather(trans2d, gid2d, zeros_g)
    return _tc_elementwise(log_beta, gt, susceptibility, uniform_noise,
                           is_infected, infection_time)
